# baseline (speedup 1.0000x reference)
import math

import jax
import jax.numpy as jnp
from jax import lax
from jax.experimental import pallas as pl
from jax.experimental.pallas import tpu as pltpu

_FACIL_RATE = 2.0
_FIRE_THRESH = 0.85
_MAX_GATE = 8.0
_C0 = math.sqrt(2.0 / math.pi)


def _gelu(x):
    return 0.5 * x * (1.0 + jnp.tanh(_C0 * (x + 0.044715 * (x * x * x))))


def _pass1_body(nblocks):
    def body(x_ref, buf_ref, maskf_ref, facil_ref, logk_ref, logs_ref,
             gate_ref, acc_ref):
        i = pl.program_id(0)
        y = _gelu(x_ref[...])
        bm = y.shape[0]
        part = jnp.sum(y.reshape(bm // 8, 8, y.shape[1]), axis=0)

        @pl.when(i == 0)
        def _():
            acc_ref[...] = part

        @pl.when(i != 0)
        def _():
            acc_ref[...] = acc_ref[...] + part

        @pl.when(i == nblocks - 1)
        def _():
            sums = jnp.sum(acc_ref[...], axis=0, keepdims=True)
            norm = jnp.sqrt(jnp.sum(sums * sums))
            q = sums / jnp.maximum(norm, 1e-12)
            sims = lax.dot_general(buf_ref[...], q,
                                   (((1,), (1,)), ((), ())),
                                   preferred_element_type=jnp.float32)
            maskf = maskf_ref[...]
            valid = maskf > 0.0
            simsm = jnp.where(valid, sims, -1.0)
            smax = jnp.max(simsm, keepdims=True)
            ids = lax.broadcasted_iota(jnp.int32, simsm.shape, 0)
            nidx = jnp.min(jnp.where(simsm == smax, ids, jnp.int32(1 << 30)),
                           keepdims=True)
            at_n = ids == nidx
            sims_at_n = jnp.sum(jnp.where(at_n, sims, 0.0), keepdims=True)
            n_valid = jnp.sum(maskf, keepdims=True)
            sum_others = jnp.sum(jnp.where(valid, sims, 0.0),
                                 keepdims=True) - sims_at_n
            mean_others = sum_others / jnp.maximum(n_valid - 1.0, 1.0)
            contrast = jnp.where(n_valid > 1.0, smax - mean_others, 0.0)
            k_gate = jnp.clip(jnp.exp(logk_ref[0, 0]), 0.01, 5.0)
            sharp = jnp.clip(jnp.exp(logs_ref[0, 0]), 0.5, 20.0)
            fire = jnp.where(smax > _FIRE_THRESH, _FACIL_RATE, 1.0)
            facil_at_n = jnp.sum(jnp.where(at_n, facil_ref[...], 0.0),
                                 keepdims=True)
            facil_level = facil_at_n * fire
            sel = 1.0 / (1.0 + jnp.exp(-sharp * contrast))
            gate = jnp.minimum(1.0 + k_gate * (facil_level - 1.0) * sel,
                               _MAX_GATE)
            gate_ref[0, 0] = gate[0, 0]

    return body


def _pass2_body(x_ref, gate_ref, o_ref):
    o_ref[...] = _gelu(x_ref[...]) * gate_ref[0, 0]


def kernel(x, log_k_gate, log_sharpness, buf, facil, mask):
    orig_shape = x.shape
    D = x.shape[-1]
    x2d = x.reshape(-1, D)
    rows = x2d.shape[0]
    bm = 1024 if rows % 1024 == 0 else rows
    nb = rows // bm
    n_buf = buf.shape[0]

    maskf = mask.astype(jnp.float32).reshape(n_buf, 1)
    facil2d = facil.reshape(n_buf, 1)
    logk2d = log_k_gate.reshape(1, 1)
    logs2d = log_sharpness.reshape(1, 1)

    gate = pl.pallas_call(
        _pass1_body(nb),
        grid=(nb,),
        in_specs=[
            pl.BlockSpec((bm, D), lambda i: (i, 0)),
            pl.BlockSpec((n_buf, D), lambda i: (0, 0)),
            pl.BlockSpec((n_buf, 1), lambda i: (0, 0)),
            pl.BlockSpec((n_buf, 1), lambda i: (0, 0)),
            pl.BlockSpec(memory_space=pltpu.SMEM),
            pl.BlockSpec(memory_space=pltpu.SMEM),
        ],
        out_specs=pl.BlockSpec(memory_space=pltpu.SMEM),
        out_shape=jax.ShapeDtypeStruct((1, 1), jnp.float32),
        scratch_shapes=[pltpu.VMEM((8, D), jnp.float32)],
        compiler_params=pltpu.CompilerParams(
            dimension_semantics=("arbitrary",)),
    )(x2d, buf, maskf, facil2d, logk2d, logs2d)

    out2d = pl.pallas_call(
        _pass2_body,
        grid=(nb,),
        in_specs=[
            pl.BlockSpec((bm, D), lambda i: (i, 0)),
            pl.BlockSpec(memory_space=pltpu.SMEM),
        ],
        out_specs=pl.BlockSpec((bm, D), lambda i: (i, 0)),
        out_shape=jax.ShapeDtypeStruct((rows, D), jnp.float32),
        compiler_params=pltpu.CompilerParams(
            dimension_semantics=("parallel",)),
    )(x2d, gate)

    return out2d.reshape(orig_shape)


# speedup vs baseline: 1.3227x; 1.3227x over previous
import math

import jax
import jax.numpy as jnp
from jax import lax
from jax.experimental import pallas as pl
from jax.experimental.pallas import tpu as pltpu

_FACIL_RATE = 2.0
_FIRE_THRESH = 0.85
_MAX_GATE = 8.0
_C0 = math.sqrt(2.0 / math.pi)
_C1 = _C0 * 0.044715


def _gelu(x):
    t = jnp.tanh(x * (_C0 + _C1 * (x * x)))
    h = 0.5 * x
    return h + h * t


def _pass1_body(nblocks):
    def body(x_ref, buf_ref, maskf_ref, facil_ref, logk_ref, logs_ref,
             gate_ref, acc_ref):
        i = pl.program_id(0)
        y = _gelu(x_ref[...])
        bm = y.shape[0]
        ones = jnp.ones((1, bm), jnp.bfloat16)
        part = lax.dot_general(ones, y.astype(jnp.bfloat16),
                               (((1,), (0,)), ((), ())),
                               preferred_element_type=jnp.float32)

        @pl.when(i == 0)
        def _():
            acc_ref[...] = part

        @pl.when(i != 0)
        def _():
            acc_ref[...] = acc_ref[...] + part

        @pl.when(i == nblocks - 1)
        def _():
            sums = acc_ref[...]
            norm = jnp.sqrt(jnp.sum(sums * sums))
            q = sums / jnp.maximum(norm, 1e-12)
            sims = lax.dot_general(buf_ref[...], q,
                                   (((1,), (1,)), ((), ())),
                                   preferred_element_type=jnp.float32)
            maskf = maskf_ref[...]
            valid = maskf > 0.0
            simsm = jnp.where(valid, sims, -1.0)
            smax = jnp.max(simsm, keepdims=True)
            ids = lax.broadcasted_iota(jnp.int32, simsm.shape, 0)
            nidx = jnp.min(jnp.where(simsm == smax, ids, jnp.int32(1 << 30)),
                           keepdims=True)
            at_n = ids == nidx
            sims_at_n = jnp.sum(jnp.where(at_n, sims, 0.0), keepdims=True)
            n_valid = jnp.sum(maskf, keepdims=True)
            sum_others = jnp.sum(jnp.where(valid, sims, 0.0),
                                 keepdims=True) - sims_at_n
            mean_others = sum_others / jnp.maximum(n_valid - 1.0, 1.0)
            contrast = jnp.where(n_valid > 1.0, smax - mean_others, 0.0)
            k_gate = jnp.clip(jnp.exp(logk_ref[0, 0]), 0.01, 5.0)
            sharp = jnp.clip(jnp.exp(logs_ref[0, 0]), 0.5, 20.0)
            fire = jnp.where(smax > _FIRE_THRESH, _FACIL_RATE, 1.0)
            facil_at_n = jnp.sum(jnp.where(at_n, facil_ref[...], 0.0),
                                 keepdims=True)
            facil_level = facil_at_n * fire
            sel = 1.0 / (1.0 + jnp.exp(-sharp * contrast))
            gate = jnp.minimum(1.0 + k_gate * (facil_level - 1.0) * sel,
                               _MAX_GATE)
            gate_ref[0, 0] = gate[0, 0]

    return body


def _pass2_body(x_ref, gate_ref, o_ref):
    o_ref[...] = _gelu(x_ref[...]) * gate_ref[0, 0]


def kernel(x, log_k_gate, log_sharpness, buf, facil, mask):
    orig_shape = x.shape
    D = x.shape[-1]
    x2d = x.reshape(-1, D)
    rows = x2d.shape[0]
    bm = 1024 if rows % 1024 == 0 else rows
    nb = rows // bm
    n_buf = buf.shape[0]

    maskf = mask.astype(jnp.float32).reshape(n_buf, 1)
    facil2d = facil.reshape(n_buf, 1)
    logk2d = log_k_gate.reshape(1, 1)
    logs2d = log_sharpness.reshape(1, 1)

    gate = pl.pallas_call(
        _pass1_body(nb),
        grid=(nb,),
        in_specs=[
            pl.BlockSpec((bm, D), lambda i: (i, 0)),
            pl.BlockSpec((n_buf, D), lambda i: (0, 0)),
            pl.BlockSpec((n_buf, 1), lambda i: (0, 0)),
            pl.BlockSpec((n_buf, 1), lambda i: (0, 0)),
            pl.BlockSpec(memory_space=pltpu.SMEM),
            pl.BlockSpec(memory_space=pltpu.SMEM),
        ],
        out_specs=pl.BlockSpec(memory_space=pltpu.SMEM),
        out_shape=jax.ShapeDtypeStruct((1, 1), jnp.float32),
        scratch_shapes=[pltpu.VMEM((1, D), jnp.float32)],
        compiler_params=pltpu.CompilerParams(
            dimension_semantics=("arbitrary",)),
    )(x2d, buf, maskf, facil2d, logk2d, logs2d)

    out2d = pl.pallas_call(
        _pass2_body,
        grid=(nb,),
        in_specs=[
            pl.BlockSpec((bm, D), lambda i: (i, 0)),
            pl.BlockSpec(memory_space=pltpu.SMEM),
        ],
        out_specs=pl.BlockSpec((bm, D), lambda i: (i, 0)),
        out_shape=jax.ShapeDtypeStruct((rows, D), jnp.float32),
        compiler_params=pltpu.CompilerParams(
            dimension_semantics=("parallel",)),
    )(x2d, gate)

    return out2d.reshape(orig_shape)
